# baseline (speedup 1.0000x reference)
import jax
import jax.numpy as jnp
from jax import lax
from jax.experimental import pallas as pl
from jax.experimental.pallas import tpu as pltpu
from jax.experimental.pallas import tpu_sc as plsc

N = 1000000
BATCH = 16384
D = 64
BLK = 32768
GRID = (N + BLK - 1) // BLK

NC = 2
NS = 16
L = 16
NW = NC * NS
BPW = BATCH // NW
CHUNK = 128
NCHUNK = BPW // CHUNK


def _sweep_body(wt_ref, b_ref, ut_ref, it_ref, pu_ref, pi_ref):
    wu = wt_ref[0:D, :]
    wi = wt_ref[D:2 * D, :]
    pu_ref[...] = jnp.sum(ut_ref[...] * wu, axis=0) + b_ref[0]
    pi_ref[...] = jnp.sum(it_ref[...] * wi, axis=0)


_SWEEP = pl.pallas_call(
    _sweep_body,
    grid=(GRID,),
    in_specs=[
        pl.BlockSpec((2 * D, 1), lambda i: (0, 0)),
        pl.BlockSpec(memory_space=pltpu.SMEM),
        pl.BlockSpec((D, BLK), lambda i: (0, i)),
        pl.BlockSpec((D, BLK), lambda i: (0, i)),
    ],
    out_specs=[
        pl.BlockSpec((BLK,), lambda i: (i,)),
        pl.BlockSpec((BLK,), lambda i: (i,)),
    ],
    out_shape=[jax.ShapeDtypeStruct((N,), jnp.float32)] * 2,
    compiler_params=pltpu.CompilerParams(dimension_semantics=("parallel",)),
)


def _gather_body(user_hbm, item_hbm, pu_hbm, pi_hbm, out_hbm,
                 idx_u, idx_i, val_u, val_i, out_v, sem_u, sem_i):
    wid = lax.axis_index("s") * NC + lax.axis_index("c")
    base = wid * BPW
    for c in range(NCHUNK):
        pltpu.sync_copy(user_hbm.at[pl.ds(base + c * CHUNK, CHUNK)], idx_u.at[c])
        pltpu.sync_copy(item_hbm.at[pl.ds(base + c * CHUNK, CHUNK)], idx_i.at[c])
    copies = []
    for c in range(NCHUNK):
        copies.append(pltpu.async_copy(pu_hbm.at[idx_u.at[c]],
                                       val_u.at[pl.ds(c * CHUNK, CHUNK)], sem_u))
        copies.append(pltpu.async_copy(pi_hbm.at[idx_i.at[c]],
                                       val_i.at[pl.ds(c * CHUNK, CHUNK)], sem_i))
    for cp in copies:
        cp.wait()
    for k in range(BPW // L):
        out_v[pl.ds(k * L, L)] = (val_u[pl.ds(k * L, L)] + val_i[pl.ds(k * L, L)])
    pltpu.sync_copy(out_v, out_hbm.at[pl.ds(base, BPW)])


_GATHER = pl.kernel(
    _gather_body,
    out_type=jax.ShapeDtypeStruct((BATCH,), jnp.float32),
    mesh=plsc.VectorSubcoreMesh(core_axis_name="c", subcore_axis_name="s"),
    compiler_params=pltpu.CompilerParams(needs_layout_passes=False,
                                         use_tc_tiling_on_sc=False),
    scratch_types=[
        pltpu.VMEM((NCHUNK, CHUNK), jnp.int32),
        pltpu.VMEM((NCHUNK, CHUNK), jnp.int32),
        pltpu.VMEM((BPW,), jnp.float32),
        pltpu.VMEM((BPW,), jnp.float32),
        pltpu.VMEM((BPW,), jnp.float32),
        pltpu.SemaphoreType.DMA,
        pltpu.SemaphoreType.DMA,
    ],
)


def kernel(user, item, user_table, item_table, W, b):
    wt = W.reshape(2 * D, 1)
    p_u, p_i = _SWEEP(wt, b, user_table.T, item_table.T)
    return _GATHER(user.astype(jnp.int32), item.astype(jnp.int32), p_u, p_i)


# speedup vs baseline: 1.0159x; 1.0159x over previous
import jax
import jax.numpy as jnp
from jax import lax
from jax.experimental import pallas as pl
from jax.experimental.pallas import tpu as pltpu
from jax.experimental.pallas import tpu_sc as plsc

N = 1000000
BATCH = 16384
D = 64
BLK = 32768
GRID = (N + BLK - 1) // BLK

NC = 2
NS = 16
L = 16
NW = NC * NS
BPW = BATCH // NW
CHUNK = 128
NCHUNK = BPW // CHUNK


def _sweep_body(w_ref, b_ref, t_ref, p_ref):
    p_ref[...] = jnp.sum(t_ref[...] * w_ref[...], axis=0) + b_ref[0]


_SWEEP = pl.pallas_call(
    _sweep_body,
    grid=(GRID,),
    in_specs=[
        pl.BlockSpec((D, 1), lambda i: (0, 0)),
        pl.BlockSpec(memory_space=pltpu.SMEM),
        pl.BlockSpec((D, BLK), lambda i: (0, i)),
    ],
    out_specs=pl.BlockSpec((BLK,), lambda i: (i,)),
    out_shape=jax.ShapeDtypeStruct((N,), jnp.float32),
    compiler_params=pltpu.CompilerParams(dimension_semantics=("parallel",)),
)


def _gather_u_body(user_hbm, pu_hbm, out_hbm, idx, val, sem):
    wid = lax.axis_index("s") * NC + lax.axis_index("c")
    base = wid * BPW
    for c in range(NCHUNK):
        pltpu.sync_copy(user_hbm.at[pl.ds(base + c * CHUNK, CHUNK)], idx.at[c])
    copies = []
    for c in range(NCHUNK):
        copies.append(pltpu.async_copy(pu_hbm.at[idx.at[c]],
                                       val.at[pl.ds(c * CHUNK, CHUNK)], sem))
    for cp in copies:
        cp.wait()
    pltpu.sync_copy(val, out_hbm.at[pl.ds(base, BPW)])


_GATHER_U = pl.kernel(
    _gather_u_body,
    out_type=jax.ShapeDtypeStruct((BATCH,), jnp.float32),
    mesh=plsc.VectorSubcoreMesh(core_axis_name="c", subcore_axis_name="s"),
    compiler_params=pltpu.CompilerParams(needs_layout_passes=False,
                                         use_tc_tiling_on_sc=False),
    scratch_types=[
        pltpu.VMEM((NCHUNK, CHUNK), jnp.int32),
        pltpu.VMEM((BPW,), jnp.float32),
        pltpu.SemaphoreType.DMA,
    ],
)


def _gather_i_body(item_hbm, pi_hbm, partial_hbm, out_hbm,
                   idx, val, acc, out_v, sem):
    wid = lax.axis_index("s") * NC + lax.axis_index("c")
    base = wid * BPW
    for c in range(NCHUNK):
        pltpu.sync_copy(item_hbm.at[pl.ds(base + c * CHUNK, CHUNK)], idx.at[c])
    pltpu.sync_copy(partial_hbm.at[pl.ds(base, BPW)], acc)
    copies = []
    for c in range(NCHUNK):
        copies.append(pltpu.async_copy(pi_hbm.at[idx.at[c]],
                                       val.at[pl.ds(c * CHUNK, CHUNK)], sem))
    for cp in copies:
        cp.wait()
    for k in range(BPW // L):
        out_v[pl.ds(k * L, L)] = val[pl.ds(k * L, L)] + acc[pl.ds(k * L, L)]
    pltpu.sync_copy(out_v, out_hbm.at[pl.ds(base, BPW)])


_GATHER_I = pl.kernel(
    _gather_i_body,
    out_type=jax.ShapeDtypeStruct((BATCH,), jnp.float32),
    mesh=plsc.VectorSubcoreMesh(core_axis_name="c", subcore_axis_name="s"),
    compiler_params=pltpu.CompilerParams(needs_layout_passes=False,
                                         use_tc_tiling_on_sc=False),
    scratch_types=[
        pltpu.VMEM((NCHUNK, CHUNK), jnp.int32),
        pltpu.VMEM((BPW,), jnp.float32),
        pltpu.VMEM((BPW,), jnp.float32),
        pltpu.VMEM((BPW,), jnp.float32),
        pltpu.SemaphoreType.DMA,
    ],
)


def kernel(user, item, user_table, item_table, W, b):
    w_u = W[0, :D].reshape(D, 1)
    w_i = W[0, D:].reshape(D, 1)
    zero = jnp.zeros_like(b)
    p_u = _SWEEP(w_u, b, user_table.T)
    part = _GATHER_U(user.astype(jnp.int32), p_u)
    p_i = _SWEEP(w_i, zero, item_table.T)
    return _GATHER_I(item.astype(jnp.int32), p_i, part)


# speedup vs baseline: 1.0172x; 1.0013x over previous
import jax
import jax.numpy as jnp
from jax import lax
from jax.experimental import pallas as pl
from jax.experimental.pallas import tpu as pltpu
from jax.experimental.pallas import tpu_sc as plsc

N = 1000000
BATCH = 16384
D = 64
BLK = 50176
GRID = (N + BLK - 1) // BLK

NC = 2
NS = 16
L = 16
NW = NC * NS
BPW = BATCH // NW
CHUNK = 128
NCHUNK = BPW // CHUNK


def _sweep_body(wt_ref, b_ref, ut_ref, it_ref, pu_ref, pi_ref):
    wu = wt_ref[0:D, :]
    wi = wt_ref[D:2 * D, :]
    pu_ref[...] = jnp.sum(ut_ref[...] * wu, axis=0) + b_ref[0]
    pi_ref[...] = jnp.sum(it_ref[...] * wi, axis=0)


_SWEEP = pl.pallas_call(
    _sweep_body,
    grid=(GRID,),
    in_specs=[
        pl.BlockSpec((2 * D, 1), lambda i: (0, 0)),
        pl.BlockSpec(memory_space=pltpu.SMEM),
        pl.BlockSpec((D, BLK), lambda i: (0, i)),
        pl.BlockSpec((D, BLK), lambda i: (0, i)),
    ],
    out_specs=[
        pl.BlockSpec((BLK,), lambda i: (i,)),
        pl.BlockSpec((BLK,), lambda i: (i,)),
    ],
    out_shape=[jax.ShapeDtypeStruct((N,), jnp.float32)] * 2,
    compiler_params=pltpu.CompilerParams(dimension_semantics=("parallel",),
                                         vmem_limit_bytes=100 * 1024 * 1024),
)


def _gather_body(user_hbm, item_hbm, pu_hbm, pi_hbm, out_hbm,
                 idx_u, idx_i, val_u, val_i, out_v, sem_u, sem_i):
    wid = lax.axis_index("s") * NC + lax.axis_index("c")
    base = wid * BPW
    for c in range(NCHUNK):
        pltpu.sync_copy(user_hbm.at[pl.ds(base + c * CHUNK, CHUNK)], idx_u.at[c])
        pltpu.sync_copy(item_hbm.at[pl.ds(base + c * CHUNK, CHUNK)], idx_i.at[c])
    copies = []
    for c in range(NCHUNK):
        copies.append(pltpu.async_copy(pu_hbm.at[idx_u.at[c]],
                                       val_u.at[pl.ds(c * CHUNK, CHUNK)], sem_u))
        copies.append(pltpu.async_copy(pi_hbm.at[idx_i.at[c]],
                                       val_i.at[pl.ds(c * CHUNK, CHUNK)], sem_i))
    for cp in copies:
        cp.wait()
    for k in range(BPW // L):
        out_v[pl.ds(k * L, L)] = (val_u[pl.ds(k * L, L)] + val_i[pl.ds(k * L, L)])
    pltpu.sync_copy(out_v, out_hbm.at[pl.ds(base, BPW)])


_GATHER = pl.kernel(
    _gather_body,
    out_type=jax.ShapeDtypeStruct((BATCH,), jnp.float32),
    mesh=plsc.VectorSubcoreMesh(core_axis_name="c", subcore_axis_name="s"),
    compiler_params=pltpu.CompilerParams(needs_layout_passes=False,
                                         use_tc_tiling_on_sc=False),
    scratch_types=[
        pltpu.VMEM((NCHUNK, CHUNK), jnp.int32),
        pltpu.VMEM((NCHUNK, CHUNK), jnp.int32),
        pltpu.VMEM((BPW,), jnp.float32),
        pltpu.VMEM((BPW,), jnp.float32),
        pltpu.VMEM((BPW,), jnp.float32),
        pltpu.SemaphoreType.DMA,
        pltpu.SemaphoreType.DMA,
    ],
)


def kernel(user, item, user_table, item_table, W, b):
    wt = W.reshape(2 * D, 1)
    p_u, p_i = _SWEEP(wt, b, user_table.T, item_table.T)
    return _GATHER(user.astype(jnp.int32), item.astype(jnp.int32), p_u, p_i)


# speedup vs baseline: 1.0224x; 1.0051x over previous
import jax
import jax.numpy as jnp
from jax import lax
from jax.experimental import pallas as pl
from jax.experimental.pallas import tpu as pltpu
from jax.experimental.pallas import tpu_sc as plsc

N = 1000000
BATCH = 16384
D = 64
BLK = 32768
GRID = (N + BLK - 1) // BLK

NC = 2
NS = 16
L = 16
NW = NC * NS
BPW = BATCH // NW
CHUNK = 128
NCHUNK = BPW // CHUNK


H = D // 2


def _sweep_body(wt_ref, b_ref, ut0_ref, ut1_ref, it0_ref, it1_ref,
                pu_ref, pi_ref):
    wu0 = wt_ref[0:H, :]
    wu1 = wt_ref[H:D, :]
    wi0 = wt_ref[D:D + H, :]
    wi1 = wt_ref[D + H:2 * D, :]
    pu_ref[...] = (jnp.sum(ut0_ref[...] * wu0, axis=0)
                   + jnp.sum(ut1_ref[...] * wu1, axis=0) + b_ref[0])
    pi_ref[...] = (jnp.sum(it0_ref[...] * wi0, axis=0)
                   + jnp.sum(it1_ref[...] * wi1, axis=0))


_SWEEP = pl.pallas_call(
    _sweep_body,
    grid=(GRID,),
    in_specs=[
        pl.BlockSpec((2 * D, 1), lambda i: (0, 0)),
        pl.BlockSpec(memory_space=pltpu.SMEM),
        pl.BlockSpec((H, BLK), lambda i: (0, i)),
        pl.BlockSpec((H, BLK), lambda i: (1, i)),
        pl.BlockSpec((H, BLK), lambda i: (0, i)),
        pl.BlockSpec((H, BLK), lambda i: (1, i)),
    ],
    out_specs=[
        pl.BlockSpec((BLK,), lambda i: (i,)),
        pl.BlockSpec((BLK,), lambda i: (i,)),
    ],
    out_shape=[jax.ShapeDtypeStruct((N,), jnp.float32)] * 2,
    compiler_params=pltpu.CompilerParams(dimension_semantics=("parallel",),
                                         vmem_limit_bytes=100 * 1024 * 1024),
)


def _gather_body(user_hbm, item_hbm, pu_hbm, pi_hbm, out_hbm,
                 idx_u, idx_i, val_u, val_i, out_v, sem_u, sem_i):
    wid = lax.axis_index("s") * NC + lax.axis_index("c")
    base = wid * BPW
    for c in range(NCHUNK):
        pltpu.sync_copy(user_hbm.at[pl.ds(base + c * CHUNK, CHUNK)], idx_u.at[c])
        pltpu.sync_copy(item_hbm.at[pl.ds(base + c * CHUNK, CHUNK)], idx_i.at[c])
    copies = []
    for c in range(NCHUNK):
        copies.append(pltpu.async_copy(pu_hbm.at[idx_u.at[c]],
                                       val_u.at[pl.ds(c * CHUNK, CHUNK)], sem_u))
        copies.append(pltpu.async_copy(pi_hbm.at[idx_i.at[c]],
                                       val_i.at[pl.ds(c * CHUNK, CHUNK)], sem_i))
    for cp in copies:
        cp.wait()
    for k in range(BPW // L):
        out_v[pl.ds(k * L, L)] = (val_u[pl.ds(k * L, L)] + val_i[pl.ds(k * L, L)])
    pltpu.sync_copy(out_v, out_hbm.at[pl.ds(base, BPW)])


_GATHER = pl.kernel(
    _gather_body,
    out_type=jax.ShapeDtypeStruct((BATCH,), jnp.float32),
    mesh=plsc.VectorSubcoreMesh(core_axis_name="c", subcore_axis_name="s"),
    compiler_params=pltpu.CompilerParams(needs_layout_passes=False,
                                         use_tc_tiling_on_sc=False),
    scratch_types=[
        pltpu.VMEM((NCHUNK, CHUNK), jnp.int32),
        pltpu.VMEM((NCHUNK, CHUNK), jnp.int32),
        pltpu.VMEM((BPW,), jnp.float32),
        pltpu.VMEM((BPW,), jnp.float32),
        pltpu.VMEM((BPW,), jnp.float32),
        pltpu.SemaphoreType.DMA,
        pltpu.SemaphoreType.DMA,
    ],
)


def kernel(user, item, user_table, item_table, W, b):
    wt = W.reshape(2 * D, 1)
    ut = user_table.T
    it = item_table.T
    p_u, p_i = _SWEEP(wt, b, ut, ut, it, it)
    return _GATHER(user.astype(jnp.int32), item.astype(jnp.int32), p_u, p_i)


# speedup vs baseline: 1.0425x; 1.0197x over previous
import jax
import jax.numpy as jnp
from jax import lax
from jax.experimental import pallas as pl
from jax.experimental.pallas import tpu as pltpu
from jax.experimental.pallas import tpu_sc as plsc

N = 1000000
BATCH = 16384
D = 64
BLK = 32768
GRID = (N + BLK - 1) // BLK

NC = 2
NS = 16
L = 16
NW = NC * NS
BPW = BATCH // NW
CHUNK = 128
NCHUNK = BPW // CHUNK


def _sweep_body(wt_ref, b_ref, ut_ref, it_ref, pu_ref, pi_ref):
    wu = wt_ref[0:D, :]
    wi = wt_ref[D:2 * D, :]
    pu_ref[...] = jnp.sum(ut_ref[...] * wu, axis=0) + b_ref[0]
    pi_ref[...] = jnp.sum(it_ref[...] * wi, axis=0)


_SWEEP = pl.pallas_call(
    _sweep_body,
    grid=(GRID,),
    in_specs=[
        pl.BlockSpec((2 * D, 1), lambda i: (0, 0)),
        pl.BlockSpec(memory_space=pltpu.SMEM),
        pl.BlockSpec((D, BLK), lambda i: (0, i)),
        pl.BlockSpec((D, BLK), lambda i: (0, i)),
    ],
    out_specs=[
        pl.BlockSpec((BLK,), lambda i: (i,)),
        pl.BlockSpec((BLK,), lambda i: (i,)),
    ],
    out_shape=[jax.ShapeDtypeStruct((N,), jnp.float32)] * 2,
    compiler_params=pltpu.CompilerParams(dimension_semantics=("parallel",)),
)


def _gather_body(user_hbm, item_hbm, pu_hbm, pi_hbm, out_hbm,
                 idx_u, idx_i, val_u, val_i, out_v, sem_u, sem_i):
    wid = lax.axis_index("s") * NC + lax.axis_index("c")
    base = wid * BPW
    cu = pltpu.async_copy(user_hbm.at[pl.ds(base, BPW)], idx_u, sem_u)
    ci = pltpu.async_copy(item_hbm.at[pl.ds(base, BPW)], idx_i, sem_i)
    cu.wait()
    ci.wait()
    gu = pltpu.async_copy(pu_hbm.at[idx_u], val_u, sem_u)
    gi = pltpu.async_copy(pi_hbm.at[idx_i], val_i, sem_i)
    gu.wait()
    gi.wait()
    for k in range(BPW // L):
        out_v[pl.ds(k * L, L)] = (val_u[pl.ds(k * L, L)] + val_i[pl.ds(k * L, L)])
    pltpu.sync_copy(out_v, out_hbm.at[pl.ds(base, BPW)])


_GATHER = pl.kernel(
    _gather_body,
    out_type=jax.ShapeDtypeStruct((BATCH,), jnp.float32),
    mesh=plsc.VectorSubcoreMesh(core_axis_name="c", subcore_axis_name="s"),
    compiler_params=pltpu.CompilerParams(needs_layout_passes=False,
                                         use_tc_tiling_on_sc=False),
    scratch_types=[
        pltpu.VMEM((BPW,), jnp.int32),
        pltpu.VMEM((BPW,), jnp.int32),
        pltpu.VMEM((BPW,), jnp.float32),
        pltpu.VMEM((BPW,), jnp.float32),
        pltpu.VMEM((BPW,), jnp.float32),
        pltpu.SemaphoreType.DMA,
        pltpu.SemaphoreType.DMA,
    ],
)


def kernel(user, item, user_table, item_table, W, b):
    wt = W.reshape(2 * D, 1)
    p_u, p_i = _SWEEP(wt, b, user_table.T, item_table.T)
    return _GATHER(user.astype(jnp.int32), item.astype(jnp.int32), p_u, p_i)
